# baseline (speedup 1.0000x reference)
import functools

import jax
import jax.numpy as jnp
from jax import lax
from jax.experimental import pallas as pl
from jax.experimental.pallas import tpu as pltpu
from jax.experimental.pallas import tpu_sc as plsc

_NC = 2
_NS = 16
_NW = _NC * _NS
_TC = 8
_BW = 128
_PITCH = 129


def kernel(X, word_table, pos_table):
    B, T = X.shape
    V, D = word_table.shape
    EB, E8, BB = D // 8, 8, B // _BW
    chunks = T // _TC

    xt = X.T

    mesh = plsc.VectorSubcoreMesh(core_axis_name="c", subcore_axis_name="s")

    @functools.partial(
        pl.kernel,
        out_type=jax.ShapeDtypeStruct((T, EB, BB, E8, _BW), jnp.float32),
        mesh=mesh,
        scratch_types=[
            pltpu.VMEM((_TC, _BW), jnp.int32),
            pltpu.VMEM((_TC * _BW, D), jnp.float32),
            pltpu.VMEM((D, _BW), jnp.float32),
            pltpu.VMEM((T, D), jnp.float32),
            pltpu.SemaphoreType.DMA,
        ],
        compiler_params=pltpu.CompilerParams(
            use_tc_tiling_on_sc=False, needs_layout_passes=False
        ),
    )
    def emb(xt_hbm, tab_hbm, pos_hbm, p_hbm, idx_v, rows_v, blk_v, pos_v, sem):
        wid = lax.axis_index("s") * _NC + lax.axis_index("c")
        b0 = wid * _BW
        pltpu.sync_copy(pos_hbm, pos_v)
        lane = lax.iota(jnp.int32, 16)
        zero = jnp.zeros((16,), dtype=jnp.int32)

        def chunk_body(it, carry):
            t0 = pl.multiple_of(it * _TC, _TC)
            pltpu.sync_copy(
                xt_hbm.at[pl.ds(t0, _TC), pl.ds(b0, _BW)], idx_v
            )
            cps = [
                pltpu.async_copy(
                    tab_hbm.at[idx_v.at[ti]],
                    rows_v.at[pl.ds(ti * _BW, _BW)],
                    sem,
                )
                for ti in range(_TC)
            ]
            for cp in cps:
                cp.wait()

            def t_body(ti, c2):
                t = t0 + ti
                for e in range(D):
                    evec = jnp.full((16,), e, dtype=jnp.int32)
                    pv = plsc.load_gather(
                        pos_v.at[pl.ds(t, 1)], [zero, evec]
                    )
                    ws = [
                        plsc.load_gather(
                            rows_v.at[pl.ds(ti * _BW + 16 * k, 16)],
                            [lane, evec],
                        )
                        for k in range(_BW // 16)
                    ]
                    for k in range(_BW // 16):
                        blk_v[e, pl.ds(16 * k, 16)] = ws[k] + pv
                for eb in range(EB):
                    pltpu.sync_copy(
                        blk_v.at[pl.ds(eb * E8, E8)],
                        p_hbm.at[t, eb, wid],
                    )
                return c2

            lax.fori_loop(0, _TC, t_body, 0)
            return carry

        lax.fori_loop(0, chunks, chunk_body, 0)

    p = emb(xt, word_table, pos_table)
    return p.transpose((2, 4, 0, 1, 3)).reshape(B, T, D)


# speedup vs baseline: 1.0039x; 1.0039x over previous
import functools

import jax
import jax.numpy as jnp
from jax import lax
from jax.experimental import pallas as pl
from jax.experimental.pallas import tpu as pltpu
from jax.experimental.pallas import tpu_sc as plsc

_NC = 2
_NS = 16
_NW = _NC * _NS
_TC = 8
_BW = 128
_PITCH = 129


def kernel(X, word_table, pos_table):
    B, T = X.shape
    V, D = word_table.shape
    EB, E8, BB = D // 8, 8, B // _BW
    chunks = T // _TC

    xt = X.T

    mesh = plsc.VectorSubcoreMesh(core_axis_name="c", subcore_axis_name="s")

    @functools.partial(
        pl.kernel,
        out_type=jax.ShapeDtypeStruct((T, EB, BB, E8, _BW), jnp.float32),
        mesh=mesh,
        scratch_types=[
            pltpu.VMEM((_TC, _BW), jnp.int32),
            pltpu.VMEM((_TC * _BW, D), jnp.float32),
            pltpu.VMEM((D, _BW), jnp.float32),
            pltpu.VMEM((T, D), jnp.float32),
            pltpu.SemaphoreType.DMA,
        ],
        compiler_params=pltpu.CompilerParams(
            use_tc_tiling_on_sc=False, needs_layout_passes=False
        ),
    )
    def emb(xt_hbm, tab_hbm, pos_hbm, p_hbm, idx_v, rows_v, blk_v, pos_v, sem):
        wid = lax.axis_index("s") * _NC + lax.axis_index("c")
        b0 = wid * _BW
        pltpu.sync_copy(pos_hbm, pos_v)
        lane = lax.iota(jnp.int32, 16)
        zero = jnp.zeros((16,), dtype=jnp.int32)

        def chunk_body(it, carry):
            t0 = pl.multiple_of(it * _TC, _TC)
            pltpu.sync_copy(
                xt_hbm.at[pl.ds(t0, _TC), pl.ds(b0, _BW)], idx_v
            )
            cps = [
                pltpu.async_copy(
                    tab_hbm.at[idx_v.at[ti]],
                    rows_v.at[pl.ds(ti * _BW, _BW)],
                    sem,
                )
                for ti in range(_TC)
            ]
            for cp in cps:
                cp.wait()

            def t_body(ti, c2):
                t = t0 + ti
                tvec = jnp.full((16,), t, dtype=jnp.int32)
                bidx = [
                    lane + (ti * _BW + 16 * k) for k in range(_BW // 16)
                ]
                for e in range(D):
                    evec = jnp.full((16,), e, dtype=jnp.int32)
                    pv = plsc.load_gather(pos_v, [tvec, evec])
                    ws = [
                        plsc.load_gather(rows_v, [bidx[k], evec])
                        for k in range(_BW // 16)
                    ]
                    for k in range(_BW // 16):
                        blk_v[e, pl.ds(16 * k, 16)] = ws[k] + pv
                for eb in range(EB):
                    pltpu.sync_copy(
                        blk_v.at[pl.ds(eb * E8, E8)],
                        p_hbm.at[t, eb, wid],
                    )
                return c2

            lax.fori_loop(0, _TC, t_body, 0)
            return carry

        lax.fori_loop(0, chunks, chunk_body, 0)

    p = emb(xt, word_table, pos_table)
    return p.transpose((2, 4, 0, 1, 3)).reshape(B, T, D)


# speedup vs baseline: 2.3436x; 2.3346x over previous
import functools

import jax
import jax.numpy as jnp
from jax import lax
from jax.experimental import pallas as pl
from jax.experimental.pallas import tpu as pltpu
from jax.experimental.pallas import tpu_sc as plsc

_NC = 2
_NS = 16
_NW = _NC * _NS
_TC = 8
_BW = 128
_PITCH = 129


def kernel(X, word_table, pos_table):
    B, T = X.shape
    V, D = word_table.shape
    EB, E8, BB = D // 8, 8, B // _BW
    chunks = T // _TC

    xt = X.T

    mesh = plsc.VectorSubcoreMesh(core_axis_name="c", subcore_axis_name="s")

    @functools.partial(
        pl.kernel,
        out_type=jax.ShapeDtypeStruct((T, EB, BB, E8, _BW), jnp.float32),
        mesh=mesh,
        scratch_types=[
            pltpu.VMEM((_TC, _BW), jnp.int32),
            pltpu.VMEM((_TC * _BW, D), jnp.float32),
            pltpu.VMEM((D, _PITCH), jnp.float32),
            pltpu.VMEM((T, D), jnp.float32),
            pltpu.SemaphoreType.DMA,
        ],
        compiler_params=pltpu.CompilerParams(
            use_tc_tiling_on_sc=False, needs_layout_passes=False
        ),
    )
    def emb(xt_hbm, tab_hbm, pos_hbm, p_hbm, idx_v, rows_v, blk_v, pos_v, sem):
        wid = lax.axis_index("s") * _NC + lax.axis_index("c")
        b0 = wid * _BW
        pltpu.sync_copy(pos_hbm, pos_v)
        lane = lax.iota(jnp.int32, 16)
        evecs = [lane + 16 * c for c in range(D // 16)]

        def chunk_body(it, carry):
            t0 = pl.multiple_of(it * _TC, _TC)
            pltpu.sync_copy(
                xt_hbm.at[pl.ds(t0, _TC), pl.ds(b0, _BW)], idx_v
            )
            cps = [
                pltpu.async_copy(
                    tab_hbm.at[idx_v.at[ti]],
                    rows_v.at[pl.ds(ti * _BW, _BW)],
                    sem,
                )
                for ti in range(_TC)
            ]
            for cp in cps:
                cp.wait()

            def t_body(ti, c2):
                t = t0 + ti
                pvs = [pos_v[t, pl.ds(16 * c, 16)] for c in range(D // 16)]
                nc = D // 16
                for rp in range(_BW // 2):
                    rows = (ti * _BW + 2 * rp, ti * _BW + 2 * rp + 1)
                    bvecs = (
                        jnp.full((16,), 2 * rp, dtype=jnp.int32),
                        jnp.full((16,), 2 * rp + 1, dtype=jnp.int32),
                    )
                    ws = [
                        rows_v[rows[j], pl.ds(16 * c, 16)] + pvs[c]
                        for j in range(2)
                        for c in range(nc)
                    ]
                    for j in range(2):
                        for c in range(nc):
                            plsc.store_scatter(
                                blk_v,
                                [evecs[c], bvecs[j]],
                                ws[j * nc + c],
                            )
                for eb in range(EB):
                    pltpu.sync_copy(
                        blk_v.at[pl.ds(eb * E8, E8), pl.ds(0, _BW)],
                        p_hbm.at[t, eb, wid],
                    )
                return c2

            lax.fori_loop(0, _TC, t_body, 0)
            return carry

        lax.fori_loop(0, chunks, chunk_body, 0)

    p = emb(xt, word_table, pos_table)
    return p.transpose((2, 4, 0, 1, 3)).reshape(B, T, D)
